# baseline (speedup 1.0000x reference)
import functools

import jax
import jax.numpy as jnp
from jax import lax
from jax.experimental import pallas as pl
from jax.experimental.pallas import tpu as pltpu
from jax.experimental.pallas import tpu_sc as plsc

N = 10000
DEG = 32
D_IN = 128
D_OUT = 128

NW = 32
CH = 8
NCHUNKS = N // CH
TRIPS = -(-NCHUNKS // NW)
LANES = 16
NLANE = D_IN // LANES

_mesh = plsc.VectorSubcoreMesh(core_axis_name="c", subcore_axis_name="s")


@functools.partial(
    pl.kernel,
    out_type=jax.ShapeDtypeStruct((N, D_IN), jnp.float32),
    mesh=_mesh,
    scratch_types=[
        pltpu.VMEM((2, CH, DEG, D_IN), jnp.float32),
        pltpu.VMEM((2, CH, D_IN), jnp.float32),
        pltpu.VMEM((2, CH, D_IN), jnp.float32),
        pltpu.SemaphoreType.DMA,
        pltpu.SemaphoreType.DMA,
        pltpu.SemaphoreType.DMA,
        pltpu.SemaphoreType.DMA,
        pltpu.SemaphoreType.DMA,
        pltpu.SemaphoreType.DMA,
    ],
)
def _sc_aggregate(emb_hbm, feat_hbm, out_hbm,
                  embuf, featbuf, outbuf, es0, es1, fs0, fs1, os0, os1):
    wid = lax.axis_index("s") * 2 + lax.axis_index("c")
    es = (es0, es1)
    fs = (fs0, fs1)
    osem = (os0, os1)

    def base_of(trip):
        cid = jnp.minimum(wid + NW * trip, NCHUNKS - 1)
        return cid * CH

    def in_copies(trip, b):
        base = base_of(trip)
        return (
            pltpu.make_async_copy(
                emb_hbm.at[pl.ds(base, CH)], embuf.at[b], es[b]),
            pltpu.make_async_copy(
                feat_hbm.at[pl.ds(base, CH)], featbuf.at[b], fs[b]),
        )

    def out_copy(trip, b):
        base = base_of(trip)
        return pltpu.make_async_copy(
            outbuf.at[b], out_hbm.at[pl.ds(base, CH)], osem[b])

    for b in range(2):
        for c in in_copies(b, b):
            c.start()

    @pl.loop(0, TRIPS, step=2)
    def chunk_loop(t):
        for b in range(2):
            trip = t + b
            for c in in_copies(trip, b):
                c.wait()

            @pl.when(trip >= 2)
            def _():
                out_copy(trip - 2, b).wait()

            eb = embuf.at[b]
            fb = featbuf.at[b]
            ob = outbuf.at[b]

            @pl.loop(0, CH)
            def node_loop(n):
                for l in range(NLANE):
                    sl = pl.ds(LANES * l, LANES)
                    acc = eb[n, 0, sl]
                    for j in range(1, DEG):
                        acc = acc + eb[n, j, sl]
                    ob[n, sl] = acc * (1.0 / DEG) + fb[n, sl]

            out_copy(trip, b).start()

            @pl.when(trip + 2 < TRIPS)
            def _():
                for c in in_copies(trip + 2, b):
                    c.start()

    for b in range(2):
        out_copy(TRIPS - 2 + b, b).wait()


def _matmul_body(x_ref, w_ref, b_ref, out_ref):
    y = lax.dot_general(
        x_ref[...], w_ref[...], (((1,), (0,)), ((), ())),
        preferred_element_type=jnp.float32)
    out_ref[...] = jnp.maximum(y + b_ref[...], 0.0)


def _tc_project(agg, w, bias2d):
    block = 400
    return pl.pallas_call(
        _matmul_body,
        grid=(N // block,),
        in_specs=[
            pl.BlockSpec((block, D_IN), lambda i: (i, 0)),
            pl.BlockSpec((D_IN, D_OUT), lambda i: (0, 0)),
            pl.BlockSpec((1, D_OUT), lambda i: (0, 0)),
        ],
        out_specs=pl.BlockSpec((block, D_OUT), lambda i: (i, 0)),
        out_shape=jax.ShapeDtypeStruct((N, D_OUT), jnp.float32),
    )(agg, w, bias2d)


@jax.jit
def kernel(features, embedding_look_up, kernel, bias_weights):
    agg = _sc_aggregate(embedding_look_up, features)
    return _tc_project(agg, kernel, bias_weights.reshape(1, D_OUT))


# speedup vs baseline: 1.8550x; 1.8550x over previous
import functools

import jax
import jax.numpy as jnp
from jax import lax
from jax.experimental import pallas as pl
from jax.experimental.pallas import tpu as pltpu
from jax.experimental.pallas import tpu_sc as plsc

N = 10000
DEG = 32
D_IN = 128
D_OUT = 128

K_SC = 2400

NW = 32
CH = 8
NCHUNKS = K_SC // CH
TRIPS = -(-NCHUNKS // NW)
TRIPS += TRIPS % 2
LANES = 16
NLANE = D_IN // LANES

_mesh = plsc.VectorSubcoreMesh(core_axis_name="c", subcore_axis_name="s")


@functools.partial(
    pl.kernel,
    out_type=jax.ShapeDtypeStruct((K_SC, D_IN), jnp.float32),
    mesh=_mesh,
    scratch_types=[
        pltpu.VMEM((2, CH, DEG, D_IN), jnp.float32),
        pltpu.VMEM((2, CH, D_IN), jnp.float32),
        pltpu.VMEM((2, CH, D_IN), jnp.float32),
        pltpu.SemaphoreType.DMA,
        pltpu.SemaphoreType.DMA,
        pltpu.SemaphoreType.DMA,
        pltpu.SemaphoreType.DMA,
        pltpu.SemaphoreType.DMA,
        pltpu.SemaphoreType.DMA,
    ],
)
def _sc_aggregate(emb_hbm, feat_hbm, out_hbm,
                  embuf, featbuf, outbuf, es0, es1, fs0, fs1, os0, os1):
    wid = lax.axis_index("s") * 2 + lax.axis_index("c")
    es = (es0, es1)
    fs = (fs0, fs1)
    osem = (os0, os1)

    def base_of(trip):
        cid = jnp.minimum(wid + NW * trip, NCHUNKS - 1)
        return cid * CH

    def in_copies(trip, b):
        base = base_of(trip)
        return (
            pltpu.make_async_copy(
                emb_hbm.at[pl.ds(base, CH)], embuf.at[b], es[b]),
            pltpu.make_async_copy(
                feat_hbm.at[pl.ds(base, CH)], featbuf.at[b], fs[b]),
        )

    def out_copy(trip, b):
        base = base_of(trip)
        return pltpu.make_async_copy(
            outbuf.at[b], out_hbm.at[pl.ds(base, CH)], osem[b])

    for b in range(2):
        for c in in_copies(b, b):
            c.start()

    @pl.loop(0, TRIPS, step=2)
    def chunk_loop(t):
        for b in range(2):
            trip = t + b
            for c in in_copies(trip, b):
                c.wait()

            @pl.when(trip >= 2)
            def _():
                out_copy(trip - 2, b).wait()

            eb = embuf.at[b]
            fb = featbuf.at[b]
            ob = outbuf.at[b]

            @pl.loop(0, CH)
            def node_loop(n):
                for l in range(NLANE):
                    sl = pl.ds(LANES * l, LANES)
                    acc = eb[n, 0, sl]
                    for j in range(1, DEG):
                        acc = acc + eb[n, j, sl]
                    ob[n, sl] = acc * (1.0 / DEG) + fb[n, sl]

            out_copy(trip, b).start()

            @pl.when(trip + 2 < TRIPS)
            def _():
                for c in in_copies(trip + 2, b):
                    c.start()

    for b in range(2):
        out_copy(TRIPS - 2 + b, b).wait()


def _matmul_body(x_ref, w_ref, b_ref, out_ref):
    y = lax.dot_general(
        x_ref[...], w_ref[...], (((1,), (0,)), ((), ())),
        preferred_element_type=jnp.float32)
    out_ref[...] = jnp.maximum(y + b_ref[...], 0.0)


def _tc_project(agg, w, bias2d):
    block = 400
    return pl.pallas_call(
        _matmul_body,
        grid=(K_SC // block,),
        in_specs=[
            pl.BlockSpec((block, D_IN), lambda i: (i, 0)),
            pl.BlockSpec((D_IN, D_OUT), lambda i: (0, 0)),
            pl.BlockSpec((1, D_OUT), lambda i: (0, 0)),
        ],
        out_specs=pl.BlockSpec((block, D_OUT), lambda i: (i, 0)),
        out_shape=jax.ShapeDtypeStruct((K_SC, D_OUT), jnp.float32),
    )(agg, w, bias2d)


def _fused_body(feat_ref, emb_ref, w_ref, b_ref, out_ref):
    agg = jnp.sum(emb_ref[...], axis=1) * (1.0 / DEG)
    x = feat_ref[...] + agg
    y = lax.dot_general(
        x, w_ref[...], (((1,), (0,)), ((), ())),
        preferred_element_type=jnp.float32)
    out_ref[...] = jnp.maximum(y + b_ref[...], 0.0)


def _tc_fused_tail(features, embedding_look_up, w, bias2d):
    block = 400
    off = K_SC // block
    return pl.pallas_call(
        _fused_body,
        grid=((N - K_SC) // block,),
        in_specs=[
            pl.BlockSpec((block, D_IN), lambda i: (i + off, 0)),
            pl.BlockSpec((block, DEG, D_IN), lambda i: (i + off, 0, 0)),
            pl.BlockSpec((D_IN, D_OUT), lambda i: (0, 0)),
            pl.BlockSpec((1, D_OUT), lambda i: (0, 0)),
        ],
        out_specs=pl.BlockSpec((block, D_OUT), lambda i: (i, 0)),
        out_shape=jax.ShapeDtypeStruct((N - K_SC, D_OUT), jnp.float32),
        compiler_params=pltpu.CompilerParams(
            dimension_semantics=("arbitrary",),
        ),
    )(features, embedding_look_up, w, bias2d)


@jax.jit
def kernel(features, embedding_look_up, kernel, bias_weights):
    bias2d = bias_weights.reshape(1, D_OUT)
    agg = _sc_aggregate(embedding_look_up, features)
    tail = _tc_fused_tail(features, embedding_look_up, kernel, bias2d)
    head = _tc_project(agg, kernel, bias2d)
    return jnp.concatenate([head, tail], axis=0)


# speedup vs baseline: 2.3332x; 1.2578x over previous
import jax
import jax.numpy as jnp
from jax import lax
from jax.experimental import pallas as pl
from jax.experimental.pallas import tpu as pltpu

N = 10000
DEG = 32
D_IN = 128
D_OUT = 128
BLOCK_R = 200


def _fused_body(feat_ref, emb_ref, w_ref, b_ref, out_ref):
    agg = jnp.sum(emb_ref[...], axis=1) * (1.0 / DEG)
    x = feat_ref[...] + agg
    y = lax.dot_general(
        x, w_ref[...], (((1,), (0,)), ((), ())),
        preferred_element_type=jnp.float32)
    out_ref[...] = jnp.maximum(y + b_ref[...], 0.0)


@jax.jit
def kernel(features, embedding_look_up, kernel, bias_weights):
    bias2d = bias_weights.reshape(1, D_OUT)
    return pl.pallas_call(
        _fused_body,
        grid=(N // BLOCK_R,),
        in_specs=[
            pl.BlockSpec((BLOCK_R, D_IN), lambda i: (i, 0)),
            pl.BlockSpec((BLOCK_R, DEG, D_IN), lambda i: (i, 0, 0)),
            pl.BlockSpec((D_IN, D_OUT), lambda i: (0, 0)),
            pl.BlockSpec((1, D_OUT), lambda i: (0, 0)),
        ],
        out_specs=pl.BlockSpec((BLOCK_R, D_OUT), lambda i: (i, 0)),
        out_shape=jax.ShapeDtypeStruct((N, D_OUT), jnp.float32),
        compiler_params=pltpu.CompilerParams(
            dimension_semantics=("arbitrary",),
        ),
    )(features, embedding_look_up, kernel, bias2d)


# speedup vs baseline: 2.7717x; 1.1880x over previous
import jax
import jax.numpy as jnp
from jax import lax
from jax.experimental import pallas as pl
from jax.experimental.pallas import tpu as pltpu

N = 10000
DEG = 32
D_IN = 128
D_OUT = 128
BLOCK_R = 1000


def _fused_body(feat_ref, emb_ref, w_ref, b_ref, out_ref):
    agg = jnp.sum(emb_ref[...], axis=1) * (1.0 / DEG)
    x = feat_ref[...] + agg
    y = lax.dot_general(
        x, w_ref[...], (((1,), (0,)), ((), ())),
        preferred_element_type=jnp.float32)
    out_ref[...] = jnp.maximum(y + b_ref[...], 0.0)


@jax.jit
def kernel(features, embedding_look_up, kernel, bias_weights):
    bias2d = bias_weights.reshape(1, D_OUT)
    return pl.pallas_call(
        _fused_body,
        grid=(N // BLOCK_R,),
        in_specs=[
            pl.BlockSpec((BLOCK_R, D_IN), lambda i: (i, 0)),
            pl.BlockSpec((BLOCK_R, DEG, D_IN), lambda i: (i, 0, 0)),
            pl.BlockSpec((D_IN, D_OUT), lambda i: (0, 0)),
            pl.BlockSpec((1, D_OUT), lambda i: (0, 0)),
        ],
        out_specs=pl.BlockSpec((BLOCK_R, D_OUT), lambda i: (i, 0)),
        out_shape=jax.ShapeDtypeStruct((N, D_OUT), jnp.float32),
        compiler_params=pltpu.CompilerParams(
            dimension_semantics=("arbitrary",),
        ),
    )(features, embedding_look_up, kernel, bias2d)


# speedup vs baseline: 2.8543x; 1.0298x over previous
import jax
import jax.numpy as jnp
from jax import lax
from jax.experimental import pallas as pl
from jax.experimental.pallas import tpu as pltpu

N = 10000
DEG = 32
D_IN = 128
D_OUT = 128
BLOCK_R = 512


def _fused_body(feat_ref, emb_ref, w_ref, b_ref, out_ref):
    agg = jnp.sum(emb_ref[...], axis=1) * (1.0 / DEG)
    x = feat_ref[...] + agg
    y = lax.dot_general(
        x, w_ref[...], (((1,), (0,)), ((), ())),
        preferred_element_type=jnp.float32)
    out_ref[...] = jnp.maximum(y + b_ref[...], 0.0)


@jax.jit
def kernel(features, embedding_look_up, kernel, bias_weights):
    bias2d = bias_weights.reshape(1, D_OUT)
    return pl.pallas_call(
        _fused_body,
        grid=(-(-N // BLOCK_R),),
        in_specs=[
            pl.BlockSpec((BLOCK_R, D_IN), lambda i: (i, 0)),
            pl.BlockSpec((BLOCK_R, DEG, D_IN), lambda i: (i, 0, 0)),
            pl.BlockSpec((D_IN, D_OUT), lambda i: (0, 0)),
            pl.BlockSpec((1, D_OUT), lambda i: (0, 0)),
        ],
        out_specs=pl.BlockSpec((BLOCK_R, D_OUT), lambda i: (i, 0)),
        out_shape=jax.ShapeDtypeStruct((N, D_OUT), jnp.float32),
        compiler_params=pltpu.CompilerParams(
            dimension_semantics=("arbitrary",),
        ),
    )(features, embedding_look_up, kernel, bias2d)
